# baseline (speedup 1.0000x reference)
import functools

import jax
import jax.numpy as jnp
from jax import lax
from jax.experimental import pallas as pl

DIM = 768
N_EXPERTS = 64
TOP_K = 8
BLOCK_T = 4096

_IDX_MASK = N_EXPERTS - 1


def _gate_block(x_ref, w_ref, b_ref, wout_ref, iout_ref):
    x = x_ref[...]
    w = w_ref[...]
    scores = jnp.dot(x, w, preferred_element_type=jnp.float32) + b_ref[...]
    eiota = lax.broadcasted_iota(jnp.int32, scores.shape, 1)
    oiota = lax.broadcasted_iota(jnp.int32, (scores.shape[0], TOP_K), 1)
    svals = jnp.zeros((scores.shape[0], TOP_K), jnp.float32)
    sidx = jnp.zeros((scores.shape[0], TOP_K), jnp.int32)
    m0 = None
    for k in range(TOP_K):
        m = jnp.max(scores, axis=-1, keepdims=True)
        idx = jnp.argmax(scores, axis=-1)[:, None]
        scores = jnp.where(eiota == idx, -jnp.inf, scores)
        svals = jnp.where(oiota == k, m, svals)
        sidx = jnp.where(oiota == k, idx, sidx)
        if k == 0:
            m0 = m
    iout_ref[...] = sidx
    e = jnp.exp(svals - m0)
    wout_ref[...] = e / (jnp.sum(e, axis=-1, keepdims=True) + 1e-8)


@functools.partial(jax.jit, static_argnames=())
def kernel(x, gate_weight, adaptive_bias):
    orig_shape = x.shape
    xf = x.reshape(-1, orig_shape[-1])
    t = xf.shape[0]
    bt = min(BLOCK_T, t)
    wt = gate_weight.T
    bias = adaptive_bias.reshape(1, N_EXPERTS)
    grid = (pl.cdiv(t, bt),)
    wts, idx = pl.pallas_call(
        _gate_block,
        grid=grid,
        in_specs=[
            pl.BlockSpec((bt, DIM), lambda i: (i, 0)),
            pl.BlockSpec((DIM, N_EXPERTS), lambda i: (0, 0)),
            pl.BlockSpec((1, N_EXPERTS), lambda i: (0, 0)),
        ],
        out_specs=[
            pl.BlockSpec((bt, TOP_K), lambda i: (i, 0)),
            pl.BlockSpec((bt, TOP_K), lambda i: (i, 0)),
        ],
        out_shape=[
            jax.ShapeDtypeStruct((t, TOP_K), jnp.float32),
            jax.ShapeDtypeStruct((t, TOP_K), jnp.int32),
        ],
    )(xf, wt, bias)
    if len(orig_shape) == 3:
        wts = wts.reshape(orig_shape[0], orig_shape[1], TOP_K)
        idx = idx.reshape(orig_shape[0], orig_shape[1], TOP_K)
    return (wts, idx)


# speedup vs baseline: 2.4140x; 2.4140x over previous
import functools

import jax
import jax.numpy as jnp
from jax import lax
from jax.experimental import pallas as pl

DIM = 768
N_EXPERTS = 64
TOP_K = 8
BLOCK_T = 2048


def _gate_block(x_ref, w_ref, b_ref, wout_ref, iout_ref):
    x = x_ref[...]
    w = w_ref[...]
    st = lax.dot_general(w, x, (((1,), (1,)), ((), ())),
                         preferred_element_type=jnp.float32) + b_ref[...]
    eio = lax.broadcasted_iota(jnp.int32, st.shape, 0)
    kio = lax.broadcasted_iota(jnp.int32, (TOP_K, st.shape[1]), 0)
    svals = jnp.zeros((TOP_K, st.shape[1]), jnp.float32)
    sidx = jnp.zeros((TOP_K, st.shape[1]), jnp.int32)
    m0 = None
    for k in range(TOP_K):
        m = jnp.max(st, axis=0, keepdims=True)
        idx = jnp.min(jnp.where(st == m, eio, N_EXPERTS), axis=0,
                      keepdims=True)
        st = jnp.where(eio == idx, -jnp.inf, st)
        svals = jnp.where(kio == k, m, svals)
        sidx = jnp.where(kio == k, idx, sidx)
        if k == 0:
            m0 = m
    iout_ref[...] = sidx
    e = jnp.exp(svals - m0)
    wout_ref[...] = e / (jnp.sum(e, axis=0, keepdims=True) + 1e-8)


@functools.partial(jax.jit, static_argnames=())
def kernel(x, gate_weight, adaptive_bias):
    orig_shape = x.shape
    xf = x.reshape(-1, orig_shape[-1])
    t = xf.shape[0]
    bt = min(BLOCK_T, t)
    bias = adaptive_bias.reshape(N_EXPERTS, 1)
    grid = (pl.cdiv(t, bt),)
    wts_t, idx_t = pl.pallas_call(
        _gate_block,
        grid=grid,
        in_specs=[
            pl.BlockSpec((bt, DIM), lambda i: (i, 0)),
            pl.BlockSpec((N_EXPERTS, DIM), lambda i: (0, 0)),
            pl.BlockSpec((N_EXPERTS, 1), lambda i: (0, 0)),
        ],
        out_specs=[
            pl.BlockSpec((TOP_K, bt), lambda i: (0, i)),
            pl.BlockSpec((TOP_K, bt), lambda i: (0, i)),
        ],
        out_shape=[
            jax.ShapeDtypeStruct((TOP_K, t), jnp.float32),
            jax.ShapeDtypeStruct((TOP_K, t), jnp.int32),
        ],
    )(xf, gate_weight, bias)
    wts = wts_t.T
    idx = idx_t.T
    if len(orig_shape) == 3:
        wts = wts.reshape(orig_shape[0], orig_shape[1], TOP_K)
        idx = idx.reshape(orig_shape[0], orig_shape[1], TOP_K)
    return (wts, idx)


# speedup vs baseline: 2.6759x; 1.1085x over previous
import functools

import jax
import jax.numpy as jnp
from jax import lax
from jax.experimental import pallas as pl

DIM = 768
N_EXPERTS = 64
TOP_K = 8
BLOCK_T = 4096


def _gate_block(x_ref, w_ref, b_ref, wout_ref, iout_ref):
    x = x_ref[...]
    w = w_ref[...]
    st = lax.dot_general(w, x, (((1,), (1,)), ((), ())),
                         preferred_element_type=jnp.float32) + b_ref[...]
    eio = lax.broadcasted_iota(jnp.int32, st.shape, 0)
    kio = lax.broadcasted_iota(jnp.int32, (TOP_K, st.shape[1]), 0)
    svals = jnp.zeros((TOP_K, st.shape[1]), jnp.float32)
    sidx = jnp.zeros((TOP_K, st.shape[1]), jnp.int32)
    m0 = None
    for k in range(TOP_K):
        m = jnp.max(st, axis=0, keepdims=True)
        idx = jnp.min(jnp.where(st == m, eio, N_EXPERTS), axis=0,
                      keepdims=True)
        st = jnp.where(eio == idx, -jnp.inf, st)
        svals = jnp.where(kio == k, m, svals)
        sidx = jnp.where(kio == k, idx, sidx)
        if k == 0:
            m0 = m
    iout_ref[...] = sidx
    e = jnp.exp(svals - m0)
    wout_ref[...] = e / (jnp.sum(e, axis=0, keepdims=True) + 1e-8)


@functools.partial(jax.jit, static_argnames=())
def kernel(x, gate_weight, adaptive_bias):
    orig_shape = x.shape
    xf = x.reshape(-1, orig_shape[-1])
    t = xf.shape[0]
    bt = min(BLOCK_T, t)
    bias = adaptive_bias.reshape(N_EXPERTS, 1)
    grid = (pl.cdiv(t, bt),)
    wts_t, idx_t = pl.pallas_call(
        _gate_block,
        grid=grid,
        in_specs=[
            pl.BlockSpec((bt, DIM), lambda i: (i, 0)),
            pl.BlockSpec((N_EXPERTS, DIM), lambda i: (0, 0)),
            pl.BlockSpec((N_EXPERTS, 1), lambda i: (0, 0)),
        ],
        out_specs=[
            pl.BlockSpec((TOP_K, bt), lambda i: (0, i)),
            pl.BlockSpec((TOP_K, bt), lambda i: (0, i)),
        ],
        out_shape=[
            jax.ShapeDtypeStruct((TOP_K, t), jnp.float32),
            jax.ShapeDtypeStruct((TOP_K, t), jnp.int32),
        ],
    )(xf, gate_weight, bias)
    wts = wts_t.T
    idx = idx_t.T
    if len(orig_shape) == 3:
        wts = wts.reshape(orig_shape[0], orig_shape[1], TOP_K)
        idx = idx.reshape(orig_shape[0], orig_shape[1], TOP_K)
    return (wts, idx)
